# baseline (speedup 1.0000x reference)
import functools

import jax
import jax.numpy as jnp
from jax import lax
from jax.experimental import pallas as pl
from jax.experimental.pallas import tpu as pltpu
from jax.experimental.pallas import tpu_sc as plsc

PI = 3.141592653589793
GAMMA = 9.0
EMB_RANGE = 0.34375
EMB_RANGE_TYPE = 0.34375

B = 1024
D = 32
HD = D // 2
NUM_TYPE_ROWS = 1000

NC = 2
NS = 16
NW = NC * NS
B_PER_W = B // NW

BI = 128
GRID_I = B // BI


def _sc_gather(ent, ent_tableT):
    mesh = plsc.VectorSubcoreMesh(
        core_axis_name="c", subcore_axis_name="s", num_cores=NC, num_subcores=NS
    )

    @functools.partial(
        pl.kernel,
        mesh=mesh,
        out_type=jax.ShapeDtypeStruct((B, D), jnp.float32),
        scratch_types=[
            pltpu.VMEM((B_PER_W,), jnp.int32),
            pltpu.VMEM((4, 4, D, 128), jnp.float32),
            pltpu.VMEM((B_PER_W, D), jnp.float32),
            pltpu.SemaphoreType.DMA,
            pltpu.SemaphoreType.DMA,
            pltpu.SemaphoreType.DMA,
            pltpu.SemaphoreType.DMA,
        ],
        compiler_params=pltpu.CompilerParams(needs_layout_passes=False),
    )
    def gather_kernel(ent_hbm, etabT_hbm, e_out,
                      eidx_v, eslab_v, erow_v, sem0, sem1, sem2, sem3):
        wid = lax.axis_index("s") * NC + lax.axis_index("c")
        base = wid * B_PER_W
        pltpu.sync_copy(ent_hbm.at[pl.ds(base, B_PER_W)], eidx_v)

        rows_re = lax.iota(jnp.int32, 16)
        rows_im = rows_re + HD
        evec = [eidx_v[0:16], eidx_v[16:32]]
        sems = [sem0, sem1, sem2, sem3]

        def fire(chunk):
            cps = []
            for s in range(4):
                m = chunk * 4 + s
                ve = pl.multiple_of((evec[m // 16][m % 16] // 128) * 128, 128)
                cps.append(pltpu.async_copy(
                    etabT_hbm.at[:, pl.ds(ve, 128)],
                    eslab_v.at[chunk % 4, s], sems[chunk % 4]))
            return cps

        nchunks = B_PER_W // 4
        pending = [fire(0), fire(1), fire(2)]
        for chunk in range(nchunks):
            if chunk + 3 < nchunks:
                pending.append(fire(chunk + 3))
            for c in pending.pop(0):
                c.wait()
            for s in range(4):
                m = chunk * 4 + s
                lane_e = jnp.full((16,), evec[m // 16][m % 16] % 128, jnp.int32)
                slab = eslab_v.at[chunk % 4, s]
                erow_v[m, 0:HD] = plsc.load_gather(slab, [rows_re, lane_e])
                erow_v[m, HD:D] = plsc.load_gather(slab, [rows_im, lane_e])

        pltpu.sync_copy(erow_v, e_out.at[pl.ds(base, B_PER_W)])

    return gather_kernel(ent, ent_tableT)


def _score_body(mw_ref, pw_ref, tidx_ref, ttabT_ref, e_ref, out_ref,
                modT_s, ph_s):
    i = pl.program_id(0)
    scale_e = PI / EMB_RANGE
    scale_t = PI / EMB_RANGE_TYPE
    mw = mw_ref[0, 0]
    pw = pw_ref[0, 0]

    @pl.when(i == 0)
    def _prologue():
        idx_row = jnp.concatenate(
            [tidx_ref[s:s + 1, :] for s in range(8)], axis=1)
        rsel = lax.broadcasted_iota(jnp.int32, (NUM_TYPE_ROWS, B), 0)
        onehot = jnp.where(rsel == idx_row, 1.0, 0.0).astype(jnp.float32)
        tT = lax.dot_general(
            ttabT_ref[...], onehot, (((1,), (0,)), ((), ())),
            preferred_element_type=jnp.float32,
        ) * scale_t

        eT = jnp.transpose(e_ref[...], (1, 0)) * scale_e
        drT = eT[:HD, :] - tT[:HD, :]
        diT = eT[HD:, :] - tT[HD:, :]
        modT_s[...] = jnp.sqrt(drT * drT + diT * diT) * mw
        ph_s[...] = (
            jnp.sum(jnp.cos(drT) * jnp.cos(diT), axis=0, keepdims=True) * pw
            - GAMMA
        )

    slab = modT_s[:, pl.ds(i * BI, BI)]
    phr = ph_s[...]
    for p in range(BI):
        col = slab[:, p:p + 1]
        out_ref[p, :, :] = col + phr


def _tc_score(mw, pw, tidx2d, ttabT, e_g):
    return pl.pallas_call(
        _score_body,
        grid=(GRID_I,),
        in_specs=[
            pl.BlockSpec((1, 1), lambda i: (0, 0)),
            pl.BlockSpec((1, 1), lambda i: (0, 0)),
            pl.BlockSpec((8, 128), lambda i: (0, 0)),
            pl.BlockSpec((D, NUM_TYPE_ROWS), lambda i: (0, 0)),
            pl.BlockSpec((B, D), lambda i: (0, 0)),
        ],
        out_specs=pl.BlockSpec((BI, HD, B), lambda i: (i, 0, 0)),
        out_shape=jax.ShapeDtypeStruct((B, HD, B), jnp.float32),
        scratch_shapes=[
            pltpu.VMEM((HD, B), jnp.float32),
            pltpu.VMEM((1, B), jnp.float32),
        ],
    )(mw, pw, tidx2d, ttabT, e_g)


def kernel(ent, type_idx, ent_table, type_table, modulus_weight, phase_weight):
    e_g = _sc_gather(ent.astype(jnp.int32), jnp.transpose(ent_table))
    out3 = _tc_score(
        modulus_weight.reshape(1, 1).astype(jnp.float32),
        phase_weight.reshape(1, 1).astype(jnp.float32),
        type_idx.astype(jnp.int32).reshape(8, 128),
        jnp.transpose(type_table),
        e_g,
    )
    return jnp.transpose(out3, (0, 2, 1))


# speedup vs baseline: 1.0109x; 1.0109x over previous
import functools

import jax
import jax.numpy as jnp
from jax import lax
from jax.experimental import pallas as pl
from jax.experimental.pallas import tpu as pltpu
from jax.experimental.pallas import tpu_sc as plsc

PI = 3.141592653589793
GAMMA = 9.0
EMB_RANGE = 0.34375
EMB_RANGE_TYPE = 0.34375

B = 1024
D = 32
HD = D // 2
NUM_TYPE_ROWS = 1000

NC = 2
NS = 16
NW = NC * NS
B_PER_W = B // NW

BI = 128
GRID_I = B // BI


def _sc_gather(ent, ent_tableT):
    mesh = plsc.VectorSubcoreMesh(
        core_axis_name="c", subcore_axis_name="s", num_cores=NC, num_subcores=NS
    )

    @functools.partial(
        pl.kernel,
        mesh=mesh,
        out_type=jax.ShapeDtypeStruct((B, D), jnp.float32),
        scratch_types=[
            pltpu.VMEM((B_PER_W,), jnp.int32),
            pltpu.VMEM((3, 8, D, 128), jnp.float32),
            pltpu.VMEM((B_PER_W, D), jnp.float32),
            pltpu.SemaphoreType.DMA,
            pltpu.SemaphoreType.DMA,
            pltpu.SemaphoreType.DMA,
        ],
        compiler_params=pltpu.CompilerParams(needs_layout_passes=False),
    )
    def gather_kernel(ent_hbm, etabT_hbm, e_out,
                      eidx_v, eslab_v, erow_v, sem0, sem1, sem2):
        wid = lax.axis_index("s") * NC + lax.axis_index("c")
        base = wid * B_PER_W
        pltpu.sync_copy(ent_hbm.at[pl.ds(base, B_PER_W)], eidx_v)

        rows_re = lax.iota(jnp.int32, 16)
        rows_im = rows_re + HD
        evec = [eidx_v[0:16], eidx_v[16:32]]
        sems = [sem0, sem1, sem2]

        def fire(chunk):
            cps = []
            for s in range(8):
                m = chunk * 8 + s
                ve = pl.multiple_of((evec[m // 16][m % 16] // 128) * 128, 128)
                cps.append(pltpu.async_copy(
                    etabT_hbm.at[:, pl.ds(ve, 128)],
                    eslab_v.at[chunk % 3, s], sems[chunk % 3]))
            return cps

        nchunks = B_PER_W // 8
        pending = [fire(0), fire(1)]
        for chunk in range(nchunks):
            if chunk + 2 < nchunks:
                pending.append(fire(chunk + 2))
            for c in pending.pop(0):
                c.wait()
            for s in range(8):
                m = chunk * 8 + s
                lane_e = jnp.full((16,), evec[m // 16][m % 16] % 128, jnp.int32)
                slab = eslab_v.at[chunk % 3, s]
                erow_v[m, 0:HD] = plsc.load_gather(slab, [rows_re, lane_e])
                erow_v[m, HD:D] = plsc.load_gather(slab, [rows_im, lane_e])

        pltpu.sync_copy(erow_v, e_out.at[pl.ds(base, B_PER_W)])

    return gather_kernel(ent, ent_tableT)


def _score_body(mw_ref, pw_ref, tidx_ref, ttabT_ref, e_ref, out_ref,
                modT_s, ph_s):
    i = pl.program_id(0)
    scale_e = PI / EMB_RANGE
    scale_t = PI / EMB_RANGE_TYPE
    mw = mw_ref[0, 0]
    pw = pw_ref[0, 0]

    @pl.when(i == 0)
    def _prologue():
        idx_row = jnp.concatenate(
            [tidx_ref[s:s + 1, :] for s in range(8)], axis=1)
        rsel = lax.broadcasted_iota(jnp.int32, (NUM_TYPE_ROWS, B), 0)
        onehot = jnp.where(rsel == idx_row, 1.0, 0.0).astype(jnp.float32)
        tT = lax.dot_general(
            ttabT_ref[...], onehot, (((1,), (0,)), ((), ())),
            preferred_element_type=jnp.float32,
        ) * scale_t

        eT = jnp.transpose(e_ref[...], (1, 0)) * scale_e
        drT = eT[:HD, :] - tT[:HD, :]
        diT = eT[HD:, :] - tT[HD:, :]
        modT_s[...] = jnp.sqrt(drT * drT + diT * diT) * mw
        ph_s[...] = (
            jnp.sum(jnp.cos(drT) * jnp.cos(diT), axis=0, keepdims=True) * pw
            - GAMMA
        )

    slab = modT_s[:, pl.ds(i * BI, BI)]
    phr = ph_s[...]
    for p in range(BI):
        col = slab[:, p:p + 1]
        out_ref[p, :, :] = col + phr


def _tc_score(mw, pw, tidx2d, ttabT, e_g):
    return pl.pallas_call(
        _score_body,
        grid=(GRID_I,),
        in_specs=[
            pl.BlockSpec((1, 1), lambda i: (0, 0)),
            pl.BlockSpec((1, 1), lambda i: (0, 0)),
            pl.BlockSpec((8, 128), lambda i: (0, 0)),
            pl.BlockSpec((D, NUM_TYPE_ROWS), lambda i: (0, 0)),
            pl.BlockSpec((B, D), lambda i: (0, 0)),
        ],
        out_specs=pl.BlockSpec((BI, HD, B), lambda i: (i, 0, 0)),
        out_shape=jax.ShapeDtypeStruct((B, HD, B), jnp.float32),
        scratch_shapes=[
            pltpu.VMEM((HD, B), jnp.float32),
            pltpu.VMEM((1, B), jnp.float32),
        ],
    )(mw, pw, tidx2d, ttabT, e_g)


def kernel(ent, type_idx, ent_table, type_table, modulus_weight, phase_weight):
    e_g = _sc_gather(ent.astype(jnp.int32), jnp.transpose(ent_table))
    out3 = _tc_score(
        modulus_weight.reshape(1, 1).astype(jnp.float32),
        phase_weight.reshape(1, 1).astype(jnp.float32),
        type_idx.astype(jnp.int32).reshape(8, 128),
        jnp.transpose(type_table),
        e_g,
    )
    return jnp.transpose(out3, (0, 2, 1))
